# Initial kernel scaffold; baseline (speedup 1.0000x reference)
import jax
import jax.numpy as jnp
from jax.experimental import pallas as pl


def kernel(x, edge_index, global_edge_index, Wl1, bl1, Wr1, br1, att1, bias1, Wl2, bl2, Wr2, br2, att2, bias2, Wfc, bfc):
    raise NotImplementedError("write your pallas kernel here")


# speedup vs baseline: 15.9385x; 15.9385x over previous
import functools

import jax
import jax.numpy as jnp
from jax import lax
from jax.experimental import pallas as pl
from jax.experimental.pallas import tpu as pltpu
from jax.experimental.pallas import tpu_sc as plsc

N = 10000
E = 160000
H = 128
NCOMBO = 4
NSUB = 16
EPS = E // NSUB
B = 80
CHUNKS = EPS // B
NPS = N // NSUB
AW = H + 16


def _proj_body(x_ref, wl_ref, wr_ref, bl_ref, br_ref, xl_ref, xr_ref):
    xb = x_ref[...]
    xl_ref[...] = jnp.dot(xb, wl_ref[...], preferred_element_type=jnp.float32) + bl_ref[0]
    xr_ref[...] = jnp.dot(xb, wr_ref[...], preferred_element_type=jnp.float32) + br_ref[0]


def _project(x, wl_cat, wr_cat, bl_cat, br_cat):
    R = 1000
    grid = (N // R, NCOMBO)
    return pl.pallas_call(
        _proj_body,
        grid=grid,
        in_specs=[
            pl.BlockSpec((R, H), lambda i, j: (i, 0)),
            pl.BlockSpec((H, H), lambda i, j: (0, j)),
            pl.BlockSpec((H, H), lambda i, j: (0, j)),
            pl.BlockSpec((1, 1, H), lambda i, j: (j, 0, 0)),
            pl.BlockSpec((1, 1, H), lambda i, j: (j, 0, 0)),
        ],
        out_specs=[
            pl.BlockSpec((R, H), lambda i, j: (j * (N // R) + i, 0)),
            pl.BlockSpec((R, H), lambda i, j: (j * (N // R) + i, 0)),
        ],
        out_shape=[
            jax.ShapeDtypeStruct((NCOMBO * N, H), jnp.float32),
            jax.ShapeDtypeStruct((NCOMBO * N, H), jnp.float32),
        ],
    )(x, wl_cat, wr_cat, bl_cat, br_cat)


def _sc_body(xl_hbm, xr_hbm, src_hbm, dst_hbm, att_hbm, acc_hbm,
             src_v, dst_v, asrc_v, adst_v, xl_rows, xr_rows, out_rows,
             att_v, accum_sh, sem_l, sem_r):
    c = lax.axis_index("c")
    s = lax.axis_index("s")
    zero16 = jnp.zeros((16,), jnp.float32)

    for layer in range(2):
        combo = layer * 2 + c

        def zfill(r, carry):
            for kk in range(AW // 16):
                out_rows[r, pl.ds(16 * kk, 16)] = zero16
            return carry

        lax.fori_loop(0, B, zfill, 0)
        for z in range(NPS // B):
            pltpu.sync_copy(out_rows, accum_sh.at[pl.ds(s * NPS + z * B, B)])
        pltpu.sync_copy(out_rows.at[pl.ds(0, NPS % B)],
                        accum_sh.at[pl.ds(s * NPS + (NPS // B) * B, NPS % B)])
        pltpu.sync_copy(att_hbm.at[combo], att_v)
        plsc.subcore_barrier()

        base = layer * E + s * EPS

        def chunk_body(k, carry):
            off = base + k * B
            pltpu.sync_copy(src_hbm.at[pl.ds(off, B)], src_v)
            pltpu.sync_copy(dst_hbm.at[pl.ds(off, B)], dst_v)
            for j in range(B // 16):
                sl = pl.ds(16 * j, 16)
                asrc_v[sl] = src_v[sl] + combo * N
                adst_v[sl] = dst_v[sl] + combo * N
            pltpu.async_copy(xl_hbm.at[asrc_v], xl_rows, sem_l).wait()
            pltpu.async_copy(xr_hbm.at[adst_v], xr_rows, sem_r).wait()

            def edge_body(e, ecarry):
                acc = zero16
                for kk in range(H // 16):
                    sl = pl.ds(16 * kk, 16)
                    u = xl_rows[e, sl] + xr_rows[e, sl]
                    lr = jnp.maximum(u, 0.0) + 0.2 * jnp.minimum(u, 0.0)
                    acc = acc + lr * att_v[sl]
                logit = jnp.sum(acc)
                p = jnp.exp(jnp.full((16,), logit, jnp.float32))
                for kk in range(H // 16):
                    sl = pl.ds(16 * kk, 16)
                    out_rows[e, sl] = xl_rows[e, sl] * p
                out_rows[e, pl.ds(H, 16)] = p
                return ecarry

            lax.fori_loop(0, B, edge_body, 0)
            pltpu.sync_copy(out_rows, accum_sh.at[dst_v], add=True)
            return carry

        lax.fori_loop(0, CHUNKS, chunk_body, 0)
        plsc.subcore_barrier()

        pltpu.sync_copy(accum_sh.at[pl.ds(s * NPS, NPS)],
                        acc_hbm.at[pl.ds(combo * N + s * NPS, NPS)])
        plsc.subcore_barrier()


def _sc_aggregate(xl, xr, src_all, dst_all, att_all):
    mesh = plsc.VectorSubcoreMesh(core_axis_name="c", subcore_axis_name="s")
    k = pl.kernel(
        _sc_body,
        out_type=jax.ShapeDtypeStruct((NCOMBO * N, AW), jnp.float32),
        mesh=mesh,
        scratch_types=[
            pltpu.VMEM((B,), jnp.int32),
            pltpu.VMEM((B,), jnp.int32),
            pltpu.VMEM((B,), jnp.int32),
            pltpu.VMEM((B,), jnp.int32),
            pltpu.VMEM((B, H), jnp.float32),
            pltpu.VMEM((B, H), jnp.float32),
            pltpu.VMEM((B, AW), jnp.float32),
            pltpu.VMEM((H,), jnp.float32),
            pltpu.VMEM_SHARED((N, AW), jnp.float32),
            pltpu.SemaphoreType.DMA,
            pltpu.SemaphoreType.DMA,
        ],
        compiler_params=pltpu.CompilerParams(
            use_tc_tiling_on_sc=False, needs_layout_passes=False),
    )
    return k(xl, xr, src_all, dst_all, att_all)


def _final_body(acc_ref, bias_ref, wfc_ref, bfc_ref, out_ref):
    j = pl.program_id(1)
    a = acc_ref[:, :H]
    den = acc_ref[:, H:H + 1]
    xq = a / (den + 1e-16) + bias_ref[0]
    t = jnp.dot(xq, wfc_ref[0], preferred_element_type=jnp.float32)

    @pl.when(j == 0)
    def _():
        out_ref[...] = t + bfc_ref[...]

    @pl.when(j > 0)
    def _():
        out_ref[...] = out_ref[...] + t


def _final(acc, bias_cat, wfc3, bfc2):
    R = 1000
    grid = (N // R, NCOMBO)
    return pl.pallas_call(
        _final_body,
        grid=grid,
        in_specs=[
            pl.BlockSpec((R, AW), lambda i, j: (j * (N // R) + i, 0)),
            pl.BlockSpec((1, 1, H), lambda i, j: (j, 0, 0)),
            pl.BlockSpec((1, H, H), lambda i, j: (j, 0, 0)),
            pl.BlockSpec((1, H), lambda i, j: (0, 0)),
        ],
        out_specs=pl.BlockSpec((R, H), lambda i, j: (i, 0)),
        out_shape=jax.ShapeDtypeStruct((N, H), jnp.float32),
    )(acc, bias_cat, wfc3, bfc2)


def kernel(x, edge_index, global_edge_index,
           Wl1, bl1, Wr1, br1, att1, bias1,
           Wl2, bl2, Wr2, br2, att2, bias2,
           Wfc, bfc):
    wl_cat = jnp.concatenate([Wl1, Wl2], axis=1)
    wr_cat = jnp.concatenate([Wr1, Wr2], axis=1)
    bl_cat = jnp.concatenate([bl1, bl2]).reshape(NCOMBO, 1, H)
    br_cat = jnp.concatenate([br1, br2]).reshape(NCOMBO, 1, H)
    att_all = jnp.concatenate([att1, att2], axis=0)
    src_all = jnp.concatenate([edge_index[0], global_edge_index[0]])
    dst_all = jnp.concatenate([edge_index[1], global_edge_index[1]])
    bias_cat = jnp.concatenate([bias1, bias2]).reshape(NCOMBO, 1, H)
    wfc3 = Wfc.reshape(NCOMBO, H, H)
    bfc2 = bfc.reshape(1, H)

    xl, xr = _project(x, wl_cat, wr_cat, bl_cat, br_cat)
    acc = _sc_aggregate(xl, xr, src_all, dst_all, att_all)
    return _final(acc, bias_cat, wfc3, bfc2)
